# speedup vs baseline: 1.7894x; 1.7894x over previous
import jax
import jax.numpy as jnp
from jax.experimental import pallas as pl

_DIM = 4096
_E = 64
_K = 8
_BLOCK = 1024


def _gate_block(x_ref, w_ref, wout_ref, iout_ref):
    x = x_ref[...]
    w = w_ref[...]
    scores = jax.lax.dot_general(
        x, w, (((1,), (1,)), ((), ())),
        preferred_element_type=jnp.float32)
    m = jnp.max(scores, axis=1, keepdims=True)
    e = jnp.exp(scores - m)
    p = e / jnp.sum(e, axis=1, keepdims=True)

    lane = jax.lax.broadcasted_iota(jnp.int32, p.shape, 1)
    out_lane = jax.lax.broadcasted_iota(jnp.int32, (p.shape[0], _K), 1)
    wout = jnp.zeros((p.shape[0], _K), jnp.float32)
    iout = jnp.zeros((p.shape[0], _K), jnp.int32)
    work = p
    for k in range(_K):
        cur = jnp.max(work, axis=1, keepdims=True)
        idx = jnp.min(jnp.where(work == cur, lane, _E), axis=1,
                      keepdims=True)
        wout = jnp.where(out_lane == k, cur, wout)
        iout = jnp.where(out_lane == k, idx, iout)
        work = jnp.where(lane == idx, -jnp.inf, work)
    wout_ref[...] = wout
    iout_ref[...] = iout


def kernel(x, weight):
    n_tokens = x.shape[0]
    grid = (n_tokens // _BLOCK,)
    wout, iout = pl.pallas_call(
        _gate_block,
        grid=grid,
        in_specs=[
            pl.BlockSpec((_BLOCK, _DIM), lambda i: (i, 0)),
            pl.BlockSpec((_E, _DIM), lambda i: (0, 0)),
        ],
        out_specs=[
            pl.BlockSpec((_BLOCK, _K), lambda i: (i, 0)),
            pl.BlockSpec((_BLOCK, _K), lambda i: (i, 0)),
        ],
        out_shape=[
            jax.ShapeDtypeStruct((n_tokens, _K), jnp.float32),
            jax.ShapeDtypeStruct((n_tokens, _K), jnp.int32),
        ],
    )(x, weight)
    return wout, iout
